# baseline (speedup 1.0000x reference)
import functools

import jax
import jax.numpy as jnp
from jax import lax
from jax.experimental import pallas as pl
from jax.experimental.pallas import tpu as pltpu, tpu_sc as plsc

N = 10000
FANOUT = 32
D = 128
NVREG = D // 16

NC = 2
NS = 16
NW = NC * NS
R = 4
ROWS_MAIN = (N // NW) // R * R
REM = N - NW * ROWS_MAIN

_sc_mesh = plsc.VectorSubcoreMesh(core_axis_name="c", subcore_axis_name="s")


def _reduce_rows(buf, obuf, nrows):
    for r in range(nrows):
        for k in range(NVREG):
            acc = buf[r, 0, pl.ds(16 * k, 16)]
            for f in range(1, FANOUT):
                acc = acc + buf[r, f, pl.ds(16 * k, 16)]
            obuf[r, pl.ds(16 * k, 16)] = acc * (1.0 / FANOUT)


@functools.partial(
    pl.kernel,
    out_type=jax.ShapeDtypeStruct((N, D), jnp.float32),
    mesh=_sc_mesh,
    scratch_types=[
        pltpu.VMEM((R, FANOUT, D), jnp.float32),
        pltpu.VMEM((R, D), jnp.float32),
    ],
)
def _sc_mean(dst_hbm, agg_hbm, buf, obuf):
    wid = lax.axis_index("s") * NC + lax.axis_index("c")
    base = wid * ROWS_MAIN

    def chunk(i, carry):
        start = base + i * R
        pltpu.sync_copy(dst_hbm.at[pl.ds(start, R)], buf)
        _reduce_rows(buf, obuf, R)
        pltpu.sync_copy(obuf, agg_hbm.at[pl.ds(start, R)])
        return carry

    lax.fori_loop(0, ROWS_MAIN // R, chunk, 0)

    @pl.when(wid < REM)
    def _tail():
        start = NW * ROWS_MAIN + wid
        pltpu.sync_copy(dst_hbm.at[pl.ds(start, 1)], buf.at[pl.ds(0, 1)])
        _reduce_rows(buf, obuf, 1)
        pltpu.sync_copy(obuf.at[pl.ds(0, 1)], agg_hbm.at[pl.ds(start, 1)])


TC_BLK = 2000


def _tc_body(src_ref, agg_ref, w1_ref, w2_ref, b_ref, out_ref):
    out_ref[...] = (
        jnp.dot(src_ref[...], w1_ref[...], preferred_element_type=jnp.float32)
        + jnp.dot(agg_ref[...], w2_ref[...], preferred_element_type=jnp.float32)
        + b_ref[...]
    )


def kernel(src_feature, dst_feature, W, b):
    n = src_feature.shape[0]
    agg = _sc_mean(dst_feature)
    w1 = W[:D]
    w2 = W[D:]
    b2 = b.reshape(1, D)
    return pl.pallas_call(
        _tc_body,
        grid=(n // TC_BLK,),
        in_specs=[
            pl.BlockSpec((TC_BLK, D), lambda i: (i, 0)),
            pl.BlockSpec((TC_BLK, D), lambda i: (i, 0)),
            pl.BlockSpec((D, D), lambda i: (0, 0)),
            pl.BlockSpec((D, D), lambda i: (0, 0)),
            pl.BlockSpec((1, D), lambda i: (0, 0)),
        ],
        out_specs=pl.BlockSpec((TC_BLK, D), lambda i: (i, 0)),
        out_shape=jax.ShapeDtypeStruct((n, D), jnp.float32),
    )(src_feature, agg, w1, w2, b2)
